# baseline (speedup 1.0000x reference)
import jax
import jax.numpy as jnp
from jax.experimental import pallas as pl
from jax.experimental.pallas import tpu as pltpu

T = 2048
E = 8
D = 1024
H = 512

BT = 512


def _gate_kernel(x_ref, w1_ref, b1_ref, w2_ref, b2_ref, w3_ref, b3_ref,
                 t_ref, soft_ref, hard_ref, yt_ref):
    i = pl.program_id(0)
    nsteps = pl.num_programs(0)
    h = jnp.dot(x_ref[...], w1_ref[...], preferred_element_type=jnp.float32)
    h = jnp.maximum(h + b1_ref[...], 0.0)
    h = jnp.dot(h, w2_ref[...], preferred_element_type=jnp.float32)
    h = jnp.maximum(h + b2_ref[...], 0.0)
    logit = jnp.dot(h, w3_ref[...], preferred_element_type=jnp.float32)
    yt_ref[:, pl.ds(i * BT, BT)] = logit.reshape(BT, E).T

    @pl.when(i == nsteps - 1)
    def _epilogue():
        b3 = b3_ref[0, 0]
        inv_t = 1.0 / t_ref[0, 0]
        y = (yt_ref[...] + b3) * inv_t
        m = jnp.max(y, axis=0, keepdims=True)
        e = jnp.exp(y - m)
        s = jnp.sum(e, axis=0, keepdims=True)
        soft = e / s
        soft_ref[...] = soft
        ms = jnp.max(soft, axis=0, keepdims=True)
        ii = jax.lax.broadcasted_iota(jnp.int32, soft.shape, 0)
        win = jnp.min(jnp.where(soft == ms, ii, E), axis=0, keepdims=True)
        hard_ref[...] = jnp.where(ii == win, 1.0, 0.0).astype(jnp.float32)


def _gate_shard(x2d, W1, b1r, W2, b2r, W3, b3r, tr):
    tl = x2d.shape[0] // E
    R = BT * E
    soft, hard = pl.pallas_call(
        _gate_kernel,
        grid=(tl // BT,),
        in_specs=[
            pl.BlockSpec((R, D), lambda i: (i, 0)),
            pl.BlockSpec((D, H), lambda i: (0, 0)),
            pl.BlockSpec((1, H), lambda i: (0, 0)),
            pl.BlockSpec((H, H), lambda i: (0, 0)),
            pl.BlockSpec((1, H), lambda i: (0, 0)),
            pl.BlockSpec((H, 1), lambda i: (0, 0)),
            pl.BlockSpec((1, 1), lambda i: (0, 0)),
            pl.BlockSpec((1, 1), lambda i: (0, 0)),
        ],
        out_specs=[
            pl.BlockSpec((E, tl), lambda i: (0, 0)),
            pl.BlockSpec((E, tl), lambda i: (0, 0)),
        ],
        out_shape=[
            jax.ShapeDtypeStruct((E, tl), jnp.float32),
            jax.ShapeDtypeStruct((E, tl), jnp.float32),
        ],
        scratch_shapes=[pltpu.VMEM((E, tl), jnp.float32)],
    )(x2d, W1, b1r, W2, b2r, W3, b3r, tr)
    return soft, hard


def kernel(x_z, W1, b1, W2, b2, W3, b3, temperature):
    x2d = x_z.reshape(T * E, D)
    b1r = b1.reshape(1, H)
    b2r = b2.reshape(1, H)
    b3r = b3.reshape(1, 1)
    tr = temperature.reshape(1, 1)

    soft, hard = _gate_shard(x2d, W1, b1r, W2, b2r, W3, b3r, tr)
    return soft.T.reshape(T, E, 1), hard.T.reshape(T, E, 1)


# speedup vs baseline: 1.0185x; 1.0185x over previous
import jax
import jax.numpy as jnp
from jax.experimental import pallas as pl
from jax.experimental.pallas import tpu as pltpu

T = 2048
E = 8
D = 1024
H = 512

BT = 256


def _gate_kernel(x_ref, w1_ref, b1_ref, w2_ref, b2_ref, w3_ref, b3_ref,
                 t_ref, soft_ref, hard_ref, yt_ref):
    i = pl.program_id(0)
    nsteps = pl.num_programs(0)
    h = jnp.dot(x_ref[...], w1_ref[...], preferred_element_type=jnp.float32)
    h = jnp.maximum(h + b1_ref[...], 0.0)
    h = jnp.dot(h, w2_ref[...], preferred_element_type=jnp.float32)
    h = jnp.maximum(h + b2_ref[...], 0.0)
    logit = jnp.dot(h, w3_ref[...], preferred_element_type=jnp.float32)
    yt_ref[:, pl.ds(i * BT, BT)] = logit.reshape(BT, E).T

    @pl.when(i == nsteps - 1)
    def _epilogue():
        b3 = b3_ref[0, 0]
        inv_t = 1.0 / t_ref[0, 0]
        y = (yt_ref[...] + b3) * inv_t
        m = jnp.max(y, axis=0, keepdims=True)
        e = jnp.exp(y - m)
        s = jnp.sum(e, axis=0, keepdims=True)
        soft = e / s
        soft_ref[...] = soft
        ms = jnp.max(soft, axis=0, keepdims=True)
        ii = jax.lax.broadcasted_iota(jnp.int32, soft.shape, 0)
        win = jnp.min(jnp.where(soft == ms, ii, E), axis=0, keepdims=True)
        hard_ref[...] = jnp.where(ii == win, 1.0, 0.0).astype(jnp.float32)


def _gate_shard(x2d, W1, b1r, W2, b2r, W3, b3r, tr):
    tl = x2d.shape[0] // E
    R = BT * E
    soft, hard = pl.pallas_call(
        _gate_kernel,
        grid=(tl // BT,),
        in_specs=[
            pl.BlockSpec((R, D), lambda i: (i, 0)),
            pl.BlockSpec((D, H), lambda i: (0, 0)),
            pl.BlockSpec((1, H), lambda i: (0, 0)),
            pl.BlockSpec((H, H), lambda i: (0, 0)),
            pl.BlockSpec((1, H), lambda i: (0, 0)),
            pl.BlockSpec((H, 1), lambda i: (0, 0)),
            pl.BlockSpec((1, 1), lambda i: (0, 0)),
            pl.BlockSpec((1, 1), lambda i: (0, 0)),
        ],
        out_specs=[
            pl.BlockSpec((E, tl), lambda i: (0, 0)),
            pl.BlockSpec((E, tl), lambda i: (0, 0)),
        ],
        out_shape=[
            jax.ShapeDtypeStruct((E, tl), jnp.float32),
            jax.ShapeDtypeStruct((E, tl), jnp.float32),
        ],
        scratch_shapes=[pltpu.VMEM((E, tl), jnp.float32)],
    )(x2d, W1, b1r, W2, b2r, W3, b3r, tr)
    return soft, hard


def kernel(x_z, W1, b1, W2, b2, W3, b3, temperature):
    x2d = x_z.reshape(T * E, D)
    b1r = b1.reshape(1, H)
    b2r = b2.reshape(1, H)
    b3r = b3.reshape(1, 1)
    tr = temperature.reshape(1, 1)

    soft, hard = _gate_shard(x2d, W1, b1r, W2, b2r, W3, b3r, tr)
    return soft.T.reshape(T, E, 1), hard.T.reshape(T, E, 1)
